# baseline (speedup 1.0000x reference)
import functools

import jax
import jax.numpy as jnp
from jax import lax
from jax.experimental import pallas as pl
from jax.experimental.pallas import tpu as pltpu
from jax.experimental.pallas import tpu_sc as plsc

_NC = 2
_NS = 16
_NW = _NC * _NS
_CHUNK = 128
_XB = 2048


def _extract_body(x_ref, g_ref, out_ref):
    x = x_ref[...]
    g = g_ref[...]
    out = jnp.where(
        g == 0,
        x[:, 0:32],
        jnp.where(
            g == 1,
            x[:, 32:64],
            jnp.where(g == 2, x[:, 64:96], x[:, 96:128]),
        ),
    )
    out_ref[...] = out


def kernel(user_ids, table):
    B = user_ids.shape[0]
    V, D = table.shape
    per_row = 128 // D
    b_per_w = B // _NW
    n_chunks = b_per_w // _CHUNK

    ids = user_ids.astype(jnp.int32)
    row_ids = (ids // per_row).reshape(_NW, n_chunks, _CHUNK)
    tbl128 = table.reshape(V * D // 128, 128)

    mesh = plsc.VectorSubcoreMesh(core_axis_name="c", subcore_axis_name="s")

    @functools.partial(
        pl.kernel,
        out_type=jax.ShapeDtypeStruct((B, 128), jnp.float32),
        mesh=mesh,
        scratch_types=[
            pltpu.VMEM((n_chunks, _CHUNK), jnp.int32),
            pltpu.VMEM((b_per_w, 128), jnp.float32),
            pltpu.SemaphoreType.DMA,
        ],
    )
    def gather_kernel(ids_hbm, table_hbm, out_hbm, idx_v, rows_v, sem):
        wid = lax.axis_index("s") * _NC + lax.axis_index("c")
        pltpu.sync_copy(ids_hbm.at[wid], idx_v)

        copies = []
        for c in range(n_chunks):
            copies.append(
                pltpu.async_copy(
                    table_hbm.at[idx_v.at[c]],
                    rows_v.at[pl.ds(c * _CHUNK, _CHUNK), :],
                    sem,
                )
            )
        for cp in copies:
            cp.wait()

        pltpu.sync_copy(rows_v, out_hbm.at[pl.ds(wid * b_per_w, b_per_w)])

    rows128 = gather_kernel(row_ids, tbl128)

    gmod = (ids % per_row).reshape(B, 1)
    out = pl.pallas_call(
        _extract_body,
        grid=(B // _XB,),
        in_specs=[
            pl.BlockSpec((_XB, 128), lambda i: (i, 0)),
            pl.BlockSpec((_XB, 1), lambda i: (i, 0)),
        ],
        out_specs=pl.BlockSpec((_XB, D), lambda i: (i, 0)),
        out_shape=jax.ShapeDtypeStruct((B, D), jnp.float32),
    )(rows128, gmod)
    return out
